# Initial kernel scaffold; baseline (speedup 1.0000x reference)
import functools

import jax
import jax.numpy as jnp
from jax import lax
from jax.experimental import pallas as pl
from jax.experimental.pallas import tpu as pltpu
from jax.experimental.pallas import tpu_sc as plsc

_N = 10000
_E = 320000
_DF = 128
_DE = 16
_H = 256
_HH = 128
_L = 4
_G = 16

_F32 = jnp.float32


def _h0_body(x_ref, w_ref, b_ref, out_ref):
    h = jnp.dot(x_ref[...], w_ref[...], preferred_element_type=_F32) + b_ref[...]
    h = jnp.maximum(h, 0.0)
    out_ref[0] = h[:, :_HH]
    out_ref[1] = h[:, _HH:]


def _tc_h0(x, node_W, node_b):
    return pl.pallas_call(
        _h0_body,
        out_shape=jax.ShapeDtypeStruct((2, _N, _HH), _F32),
    )(x, node_W, node_b)


_BE = 2560


def _lea_body(eattr_ref, ew_ref, eb_ref, lw_ref, lb_ref, out_ref):
    ea = jnp.dot(eattr_ref[...], ew_ref[...], preferred_element_type=_F32) + eb_ref[...]
    ea = jnp.maximum(ea, 0.0)
    lea = jnp.dot(ea, lw_ref[0], preferred_element_type=_F32) + lb_ref[...]
    out_ref[0, 0] = lea[:, :_HH]
    out_ref[0, 1] = lea[:, _HH:]


def _tc_lea(edge_attr, edge_W, edge_b, lin_W, lin_b):
    n_e = _E // _BE
    return pl.pallas_call(
        _lea_body,
        grid=(_L, n_e),
        in_specs=[
            pl.BlockSpec((_BE, _DE), lambda l, e: (e, 0)),
            pl.BlockSpec((_DE, _H), lambda l, e: (0, 0)),
            pl.BlockSpec((1, _H), lambda l, e: (0, 0)),
            pl.BlockSpec((1, _H, _H), lambda l, e: (l, 0, 0)),
            pl.BlockSpec((1, _H), lambda l, e: (l, 0)),
        ],
        out_specs=pl.BlockSpec((1, 2, _BE, _HH), lambda l, e: (l, 0, e, 0)),
        out_shape=jax.ShapeDtypeStruct((_L, 2, _E, _HH), _F32),
    )(edge_attr, edge_W, edge_b, lin_W, lin_b)


_K = 400
_EPT = _E // 16
_NPT = _N // 16


def _sc_body(h2, lea2, src2, dst, agg2, aggsp, msg, sidx, didx):
    c = lax.axis_index("c")
    s = lax.axis_index("s")

    @pl.loop(0, _K)
    def _zero_row(e):
        for j in range(_HH // 16):
            msg[e, pl.ds(j * 16, 16)] = jnp.zeros((16,), _F32)

    pltpu.sync_copy(msg.at[pl.ds(0, _K)], aggsp.at[pl.ds(s * _NPT, _K)])
    pltpu.sync_copy(msg.at[pl.ds(0, _NPT - _K)],
                    aggsp.at[pl.ds(s * _NPT + _K, _NPT - _K)])
    plsc.subcore_barrier()

    ebase = s * _EPT

    @pl.loop(0, _EPT // _K)
    def _chunk(n):
        base = ebase + n * _K
        pltpu.sync_copy(src2.at[pl.ds(c * _E + base, _K)], sidx)
        pltpu.sync_copy(dst.at[pl.ds(base, _K)], didx)
        pltpu.sync_copy(lea2.at[pl.ds(c * _E + base, _K)], msg)
        pltpu.sync_copy(h2.at[sidx], msg, add=True)

        @pl.loop(0, _K)
        def _relu_row(e):
            for j in range(_HH // 16):
                v = msg[e, pl.ds(j * 16, 16)]
                msg[e, pl.ds(j * 16, 16)] = jnp.maximum(v, 0.0)

        pltpu.sync_copy(msg, aggsp.at[didx], add=True)

    plsc.subcore_barrier()
    pltpu.sync_copy(aggsp.at[pl.ds(s * _NPT, _NPT)],
                    agg2.at[pl.ds(c * _N + s * _NPT, _NPT)])


def _sc_layer(h2, lea2, src2, dst):
    mesh = plsc.VectorSubcoreMesh(core_axis_name="c", subcore_axis_name="s",
                                  num_cores=2, num_subcores=16)
    fn = pl.kernel(
        _sc_body,
        out_type=jax.ShapeDtypeStruct((2 * _N, _HH), _F32),
        mesh=mesh,
        scratch_types=[
            pltpu.VMEM_SHARED((_N, _HH), _F32),
            pltpu.VMEM((_K, _HH), _F32),
            pltpu.VMEM((_K,), jnp.int32),
            pltpu.VMEM((_K,), jnp.int32),
        ],
    )
    return fn(h2, lea2, src2, dst)


_BN = 2000


def _node_a_body(agg_ref, h_ref, m1_ref, b1_ref, m2_ref, b2_ref, t2_ref, sums_ref):
    i = pl.program_id(0)
    m1 = m1_ref[...]
    t_lo = agg_ref[0] + h_ref[0]
    t_hi = agg_ref[1] + h_ref[1]
    r1 = jnp.dot(t_lo, m1[:_HH], preferred_element_type=_F32)
    r1 = r1 + jnp.dot(t_hi, m1[_HH:], preferred_element_type=_F32) + b1_ref[...]
    r1 = jnp.maximum(r1, 0.0)
    t2 = jnp.dot(r1, m2_ref[...], preferred_element_type=_F32) + b2_ref[...]
    t2_ref[...] = t2
    ssum = jnp.sum(t2, axis=0, keepdims=True)
    ssq = jnp.sum(t2 * t2, axis=0, keepdims=True)
    both = jnp.concatenate([ssum, ssq], axis=0)

    @pl.when(i == 0)
    def _():
        sums_ref[...] = both

    @pl.when(i > 0)
    def _():
        sums_ref[...] = sums_ref[...] + both


def _node_b_body(t2_ref, h_ref, sums_ref, gnw_ref, gnb_ref, gnms_ref, out_ref):
    inv_n = 1.0 / _N
    mu = sums_ref[0:1] * inv_n
    m2s = sums_ref[1:2] * inv_n
    ms = gnms_ref[...]
    var = m2s - mu * mu * ms * (2.0 - ms)
    cen = t2_ref[...] - mu * ms
    t3 = gnw_ref[...] * cen * lax.rsqrt(var + 1e-5) + gnb_ref[...]
    t3 = jnp.maximum(t3, 0.0)
    out_ref[0] = h_ref[0] + t3[:, :_HH]
    out_ref[1] = h_ref[1] + t3[:, _HH:]


def _tc_node(agg, h, m1_W, m1_b, m2_W, m2_b, gn_w, gn_b, gn_ms):
    n_b = _N // _BN
    t2, sums = pl.pallas_call(
        _node_a_body,
        grid=(n_b,),
        in_specs=[
            pl.BlockSpec((2, _BN, _HH), lambda i: (0, i, 0)),
            pl.BlockSpec((2, _BN, _HH), lambda i: (0, i, 0)),
            pl.BlockSpec((_H, _H), lambda i: (0, 0)),
            pl.BlockSpec((1, _H), lambda i: (0, 0)),
            pl.BlockSpec((_H, _H), lambda i: (0, 0)),
            pl.BlockSpec((1, _H), lambda i: (0, 0)),
        ],
        out_specs=[
            pl.BlockSpec((_BN, _H), lambda i: (i, 0)),
            pl.BlockSpec((2, _H), lambda i: (0, 0)),
        ],
        out_shape=[
            jax.ShapeDtypeStruct((_N, _H), _F32),
            jax.ShapeDtypeStruct((2, _H), _F32),
        ],
    )(agg, h, m1_W, m1_b, m2_W, m2_b)

    return pl.pallas_call(
        _node_b_body,
        grid=(n_b,),
        in_specs=[
            pl.BlockSpec((_BN, _H), lambda i: (i, 0)),
            pl.BlockSpec((2, _BN, _HH), lambda i: (0, i, 0)),
            pl.BlockSpec((2, _H), lambda i: (0, 0)),
            pl.BlockSpec((1, _H), lambda i: (0, 0)),
            pl.BlockSpec((1, _H), lambda i: (0, 0)),
            pl.BlockSpec((1, _H), lambda i: (0, 0)),
        ],
        out_specs=pl.BlockSpec((2, _BN, _HH), lambda i: (0, i, 0)),
        out_shape=jax.ShapeDtypeStruct((2, _N, _HH), _F32),
    )(t2, h, sums, gn_w, gn_b, gn_ms)


def _final_body(h_ref, bcol_ref, brow_ref, gx_ref, g1_ref, g1b_ref, g2_ref,
                g2b_ref, gp_ref, gpb_ref, lnw_ref, lnb_ref, h1_ref, h1b_ref,
                h2_ref, h2b_ref, out_ref):
    hf = jnp.concatenate([h_ref[0], h_ref[1]], axis=1)
    bm = bcol_ref[...] == lax.broadcasted_iota(jnp.int32, (_N, _G), 1)
    bmT = brow_ref[...] == lax.broadcasted_iota(jnp.int32, (_G, _N), 0)
    bf = bm.astype(_F32)
    bfT = bmT.astype(_F32)

    cnt = jnp.dot(bfT, jnp.ones((_N, 1), _F32), preferred_element_type=_F32)
    hsum = jnp.dot(bfT, hf, preferred_element_type=_F32)
    h_mean = hsum / jnp.maximum(cnt, 1.0)

    g1 = g1_ref[...]
    gh = jnp.dot(h_ref[0], g1[:_HH], preferred_element_type=_F32)
    gh = gh + jnp.dot(h_ref[1], g1[_HH:], preferred_element_type=_F32) + g1b_ref[...]
    gh = jnp.maximum(gh, 0.0)
    gate = jnp.dot(gh, g2_ref[...], preferred_element_type=_F32) + g2b_ref[...]

    gm = jnp.max(jnp.where(bm, gate, -jnp.inf), axis=0, keepdims=True)
    gm = jnp.where(jnp.isfinite(gm), gm, 0.0)
    gmb = jnp.sum(bf * gm, axis=1, keepdims=True)
    eg = jnp.exp(gate - gmb)
    den = jnp.dot(bfT, eg, preferred_element_type=_F32)
    denb = jnp.dot(bf, den, preferred_element_type=_F32)
    alpha = eg / (denb + 1e-16)
    h_attn = jnp.dot(bfT, alpha * hf, preferred_element_type=_F32)

    g = jnp.dot(gx_ref[...], gp_ref[...], preferred_element_type=_F32) + gpb_ref[...]
    g = jnp.maximum(g, 0.0)

    zc = jnp.concatenate([h_mean, h_attn, g], axis=1)
    mu = jnp.mean(zc, axis=1, keepdims=True)
    var = jnp.mean((zc - mu) ** 2, axis=1, keepdims=True)
    zcn = lnw_ref[...] * (zc - mu) / jnp.sqrt(var + 1e-5) + lnb_ref[...]

    z1 = jnp.dot(zcn, h1_ref[...], preferred_element_type=_F32) + h1b_ref[...]
    z1 = jnp.maximum(z1, 0.0)
    out_ref[...] = jnp.dot(z1, h2_ref[...], preferred_element_type=_F32) + h2b_ref[...]


def _tc_final(h, batch_col, batch_row, global_x, g1_W, g1_b, g2_W, g2_b,
              gp_W, gp_b, ln_w, ln_b, h1_W, h1_b, h2_W, h2_b):
    return pl.pallas_call(
        _final_body,
        out_shape=jax.ShapeDtypeStruct((_G, 64), _F32),
    )(h, batch_col, batch_row, global_x, g1_W, g1_b, g2_W, g2_b,
      gp_W, gp_b, ln_w, ln_b, h1_W, h1_b, h2_W, h2_b)


def kernel(x, edge_index, edge_attr, batch, global_x, node_W, node_b, edge_W,
           edge_b, lin_W, lin_b, m1_W, m1_b, m2_W, m2_b, gn_w, gn_b, gn_ms,
           g1_W, g1_b, g2_W, g2_b, gp_W, gp_b, ln_w, ln_b, h1_W, h1_b, h2_W,
           h2_b):
    src = edge_index[0]
    dst = edge_index[1]
    src2 = jnp.concatenate([src, src + _N])

    h = _tc_h0(x, node_W, jnp.reshape(node_b, (1, _H)))
    lea_all = _tc_lea(edge_attr, edge_W, jnp.reshape(edge_b, (1, _H)), lin_W, lin_b)

    for i in range(_L):
        lea2 = jnp.reshape(lea_all[i], (2 * _E, _HH))
        h2 = jnp.reshape(h, (2 * _N, _HH))
        agg2 = _sc_layer(h2, lea2, src2, dst)
        agg = jnp.reshape(agg2, (2, _N, _HH))
        h = _tc_node(agg, h, m1_W[i], jnp.reshape(m1_b[i], (1, _H)),
                     m2_W[i], jnp.reshape(m2_b[i], (1, _H)),
                     jnp.reshape(gn_w[i], (1, _H)), jnp.reshape(gn_b[i], (1, _H)),
                     jnp.reshape(gn_ms[i], (1, _H)))

    out = _tc_final(h, jnp.reshape(batch, (_N, 1)), jnp.reshape(batch, (1, _N)),
                    global_x, g1_W, jnp.reshape(g1_b, (1, _HH)), g2_W,
                    jnp.reshape(g2_b, (1, 1)), gp_W, jnp.reshape(gp_b, (1, _H)),
                    jnp.reshape(ln_w, (1, 3 * _H)), jnp.reshape(ln_b, (1, 3 * _H)),
                    h1_W, jnp.reshape(h1_b, (1, _H)), h2_W, jnp.reshape(h2_b, (1, 64)))
    return out


# speedup vs baseline: 2.1600x; 2.1600x over previous
import functools

import jax
import jax.numpy as jnp
from jax import lax
from jax.experimental import pallas as pl
from jax.experimental.pallas import tpu as pltpu
from jax.experimental.pallas import tpu_sc as plsc

_N = 10000
_E = 320000
_DF = 128
_DE = 16
_H = 256
_HH = 128
_L = 4
_G = 16

_F32 = jnp.float32


def _h0_body(x_ref, w_ref, b_ref, out_ref):
    h = jnp.dot(x_ref[...], w_ref[...], preferred_element_type=_F32) + b_ref[...]
    h = jnp.maximum(h, 0.0)
    out_ref[0] = h[:, :_HH]
    out_ref[1] = h[:, _HH:]


def _tc_h0(x, node_W, node_b):
    return pl.pallas_call(
        _h0_body,
        out_shape=jax.ShapeDtypeStruct((2, _N, _HH), _F32),
    )(x, node_W, node_b)


_BE = 2560


def _lea_body(eattr_ref, ew_ref, eb_ref, lw_ref, lb_ref, out_ref):
    ea = jnp.dot(eattr_ref[...], ew_ref[...], preferred_element_type=_F32) + eb_ref[...]
    ea = jnp.maximum(ea, 0.0)
    lea = jnp.dot(ea, lw_ref[0], preferred_element_type=_F32) + lb_ref[0]
    out_ref[0, 0] = lea[:, :_HH]
    out_ref[0, 1] = lea[:, _HH:]


def _tc_lea(edge_attr, edge_W, edge_b, lin_W, lin_b):
    n_e = _E // _BE
    return pl.pallas_call(
        _lea_body,
        grid=(_L, n_e),
        in_specs=[
            pl.BlockSpec((_BE, _DE), lambda l, e: (e, 0)),
            pl.BlockSpec((_DE, _H), lambda l, e: (0, 0)),
            pl.BlockSpec((1, _H), lambda l, e: (0, 0)),
            pl.BlockSpec((1, _H, _H), lambda l, e: (l, 0, 0)),
            pl.BlockSpec((1, 1, _H), lambda l, e: (l, 0, 0)),
        ],
        out_specs=pl.BlockSpec((1, 2, _BE, _HH), lambda l, e: (l, 0, e, 0)),
        out_shape=jax.ShapeDtypeStruct((_L, 2, _E, _HH), _F32),
    )(edge_attr, edge_W, edge_b, lin_W, lin_b)


_K = 200
_EPT = _E // 16
_NP = 10240
_NPT = _NP // 16


def _sc_body(h2, lea2, src2, dst, agg2, aggsp, msg, sidx, didx):
    c = lax.axis_index("c")
    s = lax.axis_index("s")

    @pl.loop(0, _K)
    def _zero_row(e):
        for j in range(_HH // 16):
            msg[e, pl.ds(j * 16, 16)] = jnp.zeros((16,), _F32)

    for off in range(0, _NPT - _K + 1, _K):
        pltpu.sync_copy(msg.at[pl.ds(0, _K)], aggsp.at[pl.ds(s * _NPT + off, _K)])
    _rem = _NPT % _K
    if _rem:
        pltpu.sync_copy(msg.at[pl.ds(0, _rem)],
                        aggsp.at[pl.ds(s * _NPT + (_NPT - _rem), _rem)])
    plsc.subcore_barrier()

    ebase = s * _EPT

    @pl.loop(0, _EPT // _K)
    def _chunk(n):
        base = ebase + n * _K
        pltpu.sync_copy(src2.at[pl.ds(c * _E + base, _K)], sidx)
        pltpu.sync_copy(dst.at[pl.ds(base, _K)], didx)
        pltpu.sync_copy(lea2.at[pl.ds(c * _E + base, _K)], msg)
        pltpu.sync_copy(h2.at[sidx], msg, add=True)

        @pl.loop(0, _K)
        def _relu_row(e):
            for j in range(_HH // 16):
                v = msg[e, pl.ds(j * 16, 16)]
                msg[e, pl.ds(j * 16, 16)] = jnp.maximum(v, 0.0)

        pltpu.sync_copy(msg, aggsp.at[didx], add=True)

    plsc.subcore_barrier()
    pltpu.sync_copy(aggsp.at[pl.ds(s * _NPT, _NPT)],
                    agg2.at[pl.ds(c * _NP + s * _NPT, _NPT)])


def _sc_layer(h2, lea2, src2, dst):
    mesh = plsc.VectorSubcoreMesh(core_axis_name="c", subcore_axis_name="s",
                                  num_cores=2, num_subcores=16)
    fn = pl.kernel(
        _sc_body,
        out_type=jax.ShapeDtypeStruct((2 * _NP, _HH), _F32),
        mesh=mesh,
        scratch_types=[
            pltpu.VMEM_SHARED((_NP, _HH), _F32),
            pltpu.VMEM((_K, _HH), _F32),
            pltpu.VMEM((_K,), jnp.int32),
            pltpu.VMEM((_K,), jnp.int32),
        ],
    )
    return fn(h2, lea2, src2, dst)


_BN = 2000


def _node_a_body(agg_ref, h_ref, m1_ref, b1_ref, m2_ref, b2_ref, t2_ref, sums_ref):
    i = pl.program_id(0)
    m1 = m1_ref[...]
    t_lo = agg_ref[0] + h_ref[0]
    t_hi = agg_ref[1] + h_ref[1]
    r1 = jnp.dot(t_lo, m1[:_HH], preferred_element_type=_F32)
    r1 = r1 + jnp.dot(t_hi, m1[_HH:], preferred_element_type=_F32) + b1_ref[...]
    r1 = jnp.maximum(r1, 0.0)
    t2 = jnp.dot(r1, m2_ref[...], preferred_element_type=_F32) + b2_ref[...]
    t2_ref[...] = t2
    ssum = jnp.sum(t2, axis=0, keepdims=True)
    ssq = jnp.sum(t2 * t2, axis=0, keepdims=True)
    both = jnp.concatenate([ssum, ssq], axis=0)

    @pl.when(i == 0)
    def _():
        sums_ref[...] = both

    @pl.when(i > 0)
    def _():
        sums_ref[...] = sums_ref[...] + both


def _node_b_body(t2_ref, h_ref, sums_ref, gnw_ref, gnb_ref, gnms_ref, out_ref):
    inv_n = 1.0 / _N
    mu = sums_ref[0:1] * inv_n
    m2s = sums_ref[1:2] * inv_n
    ms = gnms_ref[...]
    var = m2s - mu * mu * ms * (2.0 - ms)
    cen = t2_ref[...] - mu * ms
    t3 = gnw_ref[...] * cen * lax.rsqrt(var + 1e-5) + gnb_ref[...]
    t3 = jnp.maximum(t3, 0.0)
    out_ref[0] = h_ref[0] + t3[:, :_HH]
    out_ref[1] = h_ref[1] + t3[:, _HH:]


def _tc_node(agg, h, m1_W, m1_b, m2_W, m2_b, gn_w, gn_b, gn_ms):
    n_b = _N // _BN
    t2, sums = pl.pallas_call(
        _node_a_body,
        grid=(n_b,),
        in_specs=[
            pl.BlockSpec((2, _BN, _HH), lambda i: (0, i, 0)),
            pl.BlockSpec((2, _BN, _HH), lambda i: (0, i, 0)),
            pl.BlockSpec((_H, _H), lambda i: (0, 0)),
            pl.BlockSpec((1, _H), lambda i: (0, 0)),
            pl.BlockSpec((_H, _H), lambda i: (0, 0)),
            pl.BlockSpec((1, _H), lambda i: (0, 0)),
        ],
        out_specs=[
            pl.BlockSpec((_BN, _H), lambda i: (i, 0)),
            pl.BlockSpec((2, _H), lambda i: (0, 0)),
        ],
        out_shape=[
            jax.ShapeDtypeStruct((_N, _H), _F32),
            jax.ShapeDtypeStruct((2, _H), _F32),
        ],
    )(agg, h, m1_W, m1_b, m2_W, m2_b)

    return pl.pallas_call(
        _node_b_body,
        grid=(n_b,),
        in_specs=[
            pl.BlockSpec((_BN, _H), lambda i: (i, 0)),
            pl.BlockSpec((2, _BN, _HH), lambda i: (0, i, 0)),
            pl.BlockSpec((2, _H), lambda i: (0, 0)),
            pl.BlockSpec((1, _H), lambda i: (0, 0)),
            pl.BlockSpec((1, _H), lambda i: (0, 0)),
            pl.BlockSpec((1, _H), lambda i: (0, 0)),
        ],
        out_specs=pl.BlockSpec((2, _BN, _HH), lambda i: (0, i, 0)),
        out_shape=jax.ShapeDtypeStruct((2, _N, _HH), _F32),
    )(t2, h, sums, gn_w, gn_b, gn_ms)


def _final_body(h_ref, bcol_ref, brow_ref, gx_ref, g1_ref, g1b_ref, g2_ref,
                g2b_ref, gp_ref, gpb_ref, lnw_ref, lnb_ref, h1_ref, h1b_ref,
                h2_ref, h2b_ref, out_ref):
    hf = jnp.concatenate([h_ref[0], h_ref[1]], axis=1)
    bm = bcol_ref[...] == lax.broadcasted_iota(jnp.int32, (_N, _G), 1)
    bmT = brow_ref[...] == lax.broadcasted_iota(jnp.int32, (_G, _N), 0)
    bf = bm.astype(_F32)
    bfT = bmT.astype(_F32)

    cnt = jnp.dot(bfT, jnp.ones((_N, 1), _F32), preferred_element_type=_F32)
    hsum = jnp.dot(bfT, hf, preferred_element_type=_F32)
    h_mean = hsum / jnp.maximum(cnt, 1.0)

    g1 = g1_ref[...]
    gh = jnp.dot(h_ref[0], g1[:_HH], preferred_element_type=_F32)
    gh = gh + jnp.dot(h_ref[1], g1[_HH:], preferred_element_type=_F32) + g1b_ref[...]
    gh = jnp.maximum(gh, 0.0)
    gate = jnp.dot(gh, g2_ref[...], preferred_element_type=_F32) + g2b_ref[...]

    gm = jnp.max(jnp.where(bm, gate, -jnp.inf), axis=0, keepdims=True)
    gm = jnp.where(jnp.isfinite(gm), gm, 0.0)
    gmb = jnp.sum(bf * gm, axis=1, keepdims=True)
    eg = jnp.exp(gate - gmb)
    den = jnp.dot(bfT, eg, preferred_element_type=_F32)
    denb = jnp.dot(bf, den, preferred_element_type=_F32)
    alpha = eg / (denb + 1e-16)
    h_attn = jnp.dot(bfT, alpha * hf, preferred_element_type=_F32)

    g = jnp.dot(gx_ref[...], gp_ref[...], preferred_element_type=_F32) + gpb_ref[...]
    g = jnp.maximum(g, 0.0)

    zc = jnp.concatenate([h_mean, h_attn, g], axis=1)
    mu = jnp.mean(zc, axis=1, keepdims=True)
    var = jnp.mean((zc - mu) ** 2, axis=1, keepdims=True)
    zcn = lnw_ref[...] * (zc - mu) / jnp.sqrt(var + 1e-5) + lnb_ref[...]

    z1 = jnp.dot(zcn, h1_ref[...], preferred_element_type=_F32) + h1b_ref[...]
    z1 = jnp.maximum(z1, 0.0)
    out_ref[...] = jnp.dot(z1, h2_ref[...], preferred_element_type=_F32) + h2b_ref[...]


def _tc_final(h, batch_col, batch_row, global_x, g1_W, g1_b, g2_W, g2_b,
              gp_W, gp_b, ln_w, ln_b, h1_W, h1_b, h2_W, h2_b):
    return pl.pallas_call(
        _final_body,
        out_shape=jax.ShapeDtypeStruct((_G, 64), _F32),
    )(h, batch_col, batch_row, global_x, g1_W, g1_b, g2_W, g2_b,
      gp_W, gp_b, ln_w, ln_b, h1_W, h1_b, h2_W, h2_b)


def kernel(x, edge_index, edge_attr, batch, global_x, node_W, node_b, edge_W,
           edge_b, lin_W, lin_b, m1_W, m1_b, m2_W, m2_b, gn_w, gn_b, gn_ms,
           g1_W, g1_b, g2_W, g2_b, gp_W, gp_b, ln_w, ln_b, h1_W, h1_b, h2_W,
           h2_b):
    src = edge_index[0]
    dst = edge_index[1]
    src2 = jnp.concatenate([src, src + _N])

    h = _tc_h0(x, node_W, jnp.reshape(node_b, (1, _H)))
    lea_all = _tc_lea(edge_attr, edge_W, jnp.reshape(edge_b, (1, _H)), lin_W,
                      jnp.reshape(lin_b, (_L, 1, _H)))

    for i in range(_L):
        lea2 = jnp.reshape(lea_all[i], (2 * _E, _HH))
        h2 = jnp.reshape(h, (2 * _N, _HH))
        agg2 = _sc_layer(h2, lea2, src2, dst)
        agg = jnp.reshape(agg2, (2, _NP, _HH))[:, :_N]
        h = _tc_node(agg, h, m1_W[i], jnp.reshape(m1_b[i], (1, _H)),
                     m2_W[i], jnp.reshape(m2_b[i], (1, _H)),
                     jnp.reshape(gn_w[i], (1, _H)), jnp.reshape(gn_b[i], (1, _H)),
                     jnp.reshape(gn_ms[i], (1, _H)))

    out = _tc_final(h, jnp.reshape(batch, (_N, 1)), jnp.reshape(batch, (1, _N)),
                    global_x, g1_W, jnp.reshape(g1_b, (1, _HH)), g2_W,
                    jnp.reshape(g2_b, (1, 1)), gp_W, jnp.reshape(gp_b, (1, _H)),
                    jnp.reshape(ln_w, (1, 3 * _H)), jnp.reshape(ln_b, (1, 3 * _H)),
                    h1_W, jnp.reshape(h1_b, (1, _H)), h2_W, jnp.reshape(h2_b, (1, 64)))
    return out


# speedup vs baseline: 3.0251x; 1.4005x over previous
import functools

import jax
import jax.numpy as jnp
from jax import lax
from jax.experimental import pallas as pl
from jax.experimental.pallas import tpu as pltpu
from jax.experimental.pallas import tpu_sc as plsc

_N = 10000
_E = 320000
_DF = 128
_DE = 16
_H = 256
_HH = 128
_L = 4
_G = 16

_F32 = jnp.float32


def _h0_body(x_ref, w_ref, b_ref, out_ref):
    h = jnp.dot(x_ref[...], w_ref[...], preferred_element_type=_F32) + b_ref[...]
    h = jnp.maximum(h, 0.0)
    out_ref[0] = h[:, :_HH]
    out_ref[1] = h[:, _HH:]


def _tc_h0(x, node_W, node_b):
    return pl.pallas_call(
        _h0_body,
        out_shape=jax.ShapeDtypeStruct((2, _N, _HH), _F32),
    )(x, node_W, node_b)


_BE = 2560


def _lea_body(eattr_ref, ew_ref, eb_ref, lw_ref, lb_ref, out_ref):
    ea = jnp.dot(eattr_ref[...], ew_ref[...], preferred_element_type=_F32) + eb_ref[...]
    ea = jnp.maximum(ea, 0.0)
    lea = jnp.dot(ea.astype(jnp.bfloat16), lw_ref[...],
                  preferred_element_type=_F32) + lb_ref[...]
    out_ref[0] = lea[:, :_HH]
    out_ref[1] = lea[:, _HH:]


def _tc_lea(edge_attr, edge_W, edge_b, lin_W_bf, lin_b):
    n_e = _E // _BE
    return pl.pallas_call(
        _lea_body,
        grid=(n_e,),
        in_specs=[
            pl.BlockSpec((_BE, _DE), lambda e: (e, 0)),
            pl.BlockSpec((_DE, _H), lambda e: (0, 0)),
            pl.BlockSpec((1, _H), lambda e: (0, 0)),
            pl.BlockSpec((_H, _H), lambda e: (0, 0)),
            pl.BlockSpec((1, _H), lambda e: (0, 0)),
        ],
        out_specs=pl.BlockSpec((2, _BE, _HH), lambda e: (0, e, 0)),
        out_shape=jax.ShapeDtypeStruct((2, _E, _HH), _F32),
    )(edge_attr, edge_W, edge_b, lin_W_bf, lin_b)


_K = 200
_EPT = _E // 16
_NP = 10240
_NPT = _NP // 16


def _sc_body(h2, lea2, src2, dst, agg2, aggsp, msg, sidx, didx):
    c = lax.axis_index("c")
    s = lax.axis_index("s")

    @pl.loop(0, _K)
    def _zero_row(e):
        for j in range(_HH // 16):
            msg[e, pl.ds(j * 16, 16)] = jnp.zeros((16,), _F32)

    for off in range(0, _NPT - _K + 1, _K):
        pltpu.sync_copy(msg.at[pl.ds(0, _K)], aggsp.at[pl.ds(s * _NPT + off, _K)])
    _rem = _NPT % _K
    if _rem:
        pltpu.sync_copy(msg.at[pl.ds(0, _rem)],
                        aggsp.at[pl.ds(s * _NPT + (_NPT - _rem), _rem)])
    plsc.subcore_barrier()

    ebase = s * _EPT

    @pl.loop(0, _EPT // _K)
    def _chunk(n):
        base = ebase + n * _K
        pltpu.sync_copy(src2.at[pl.ds(c * _E + base, _K)], sidx)
        pltpu.sync_copy(dst.at[pl.ds(base, _K)], didx)
        pltpu.sync_copy(lea2.at[pl.ds(c * _E + base, _K)], msg)
        pltpu.sync_copy(h2.at[sidx], msg, add=True)

        @pl.loop(0, _K)
        def _relu_row(e):
            for j in range(_HH // 16):
                v = msg[e, pl.ds(j * 16, 16)]
                msg[e, pl.ds(j * 16, 16)] = jnp.maximum(v, 0.0)

        pltpu.sync_copy(msg, aggsp.at[didx], add=True)

    plsc.subcore_barrier()
    pltpu.sync_copy(aggsp.at[pl.ds(s * _NPT, _NPT)],
                    agg2.at[pl.ds(c * _NP + s * _NPT, _NPT)])


def _sc_layer(h2, lea2, src2, dst):
    mesh = plsc.VectorSubcoreMesh(core_axis_name="c", subcore_axis_name="s",
                                  num_cores=2, num_subcores=16)
    fn = pl.kernel(
        _sc_body,
        out_type=jax.ShapeDtypeStruct((2 * _NP, _HH), _F32),
        mesh=mesh,
        scratch_types=[
            pltpu.VMEM_SHARED((_NP, _HH), _F32),
            pltpu.VMEM((_K, _HH), _F32),
            pltpu.VMEM((_K,), jnp.int32),
            pltpu.VMEM((_K,), jnp.int32),
        ],
    )
    return fn(h2, lea2, src2, dst)


_BN = 2000


def _node_a_body(agg_ref, h_ref, m1_ref, b1_ref, m2_ref, b2_ref, t2_ref, sums_ref):
    i = pl.program_id(0)
    m1 = m1_ref[...]
    t_lo = agg_ref[0] + h_ref[0]
    t_hi = agg_ref[1] + h_ref[1]
    r1 = jnp.dot(t_lo, m1[:_HH], preferred_element_type=_F32)
    r1 = r1 + jnp.dot(t_hi, m1[_HH:], preferred_element_type=_F32) + b1_ref[...]
    r1 = jnp.maximum(r1, 0.0)
    t2 = jnp.dot(r1, m2_ref[...], preferred_element_type=_F32) + b2_ref[...]
    t2_ref[...] = t2
    ssum = jnp.sum(t2, axis=0, keepdims=True)
    ssq = jnp.sum(t2 * t2, axis=0, keepdims=True)
    both = jnp.concatenate([ssum, ssq], axis=0)

    @pl.when(i == 0)
    def _():
        sums_ref[...] = both

    @pl.when(i > 0)
    def _():
        sums_ref[...] = sums_ref[...] + both


def _node_b_body(t2_ref, h_ref, sums_ref, gnw_ref, gnb_ref, gnms_ref, out_ref):
    inv_n = 1.0 / _N
    mu = sums_ref[0:1] * inv_n
    m2s = sums_ref[1:2] * inv_n
    ms = gnms_ref[...]
    var = m2s - mu * mu * ms * (2.0 - ms)
    cen = t2_ref[...] - mu * ms
    t3 = gnw_ref[...] * cen * lax.rsqrt(var + 1e-5) + gnb_ref[...]
    t3 = jnp.maximum(t3, 0.0)
    out_ref[0] = h_ref[0] + t3[:, :_HH]
    out_ref[1] = h_ref[1] + t3[:, _HH:]


def _tc_node(agg, h, m1_W, m1_b, m2_W, m2_b, gn_w, gn_b, gn_ms):
    n_b = _N // _BN
    t2, sums = pl.pallas_call(
        _node_a_body,
        grid=(n_b,),
        in_specs=[
            pl.BlockSpec((2, _BN, _HH), lambda i: (0, i, 0)),
            pl.BlockSpec((2, _BN, _HH), lambda i: (0, i, 0)),
            pl.BlockSpec((_H, _H), lambda i: (0, 0)),
            pl.BlockSpec((1, _H), lambda i: (0, 0)),
            pl.BlockSpec((_H, _H), lambda i: (0, 0)),
            pl.BlockSpec((1, _H), lambda i: (0, 0)),
        ],
        out_specs=[
            pl.BlockSpec((_BN, _H), lambda i: (i, 0)),
            pl.BlockSpec((2, _H), lambda i: (0, 0)),
        ],
        out_shape=[
            jax.ShapeDtypeStruct((_N, _H), _F32),
            jax.ShapeDtypeStruct((2, _H), _F32),
        ],
    )(agg, h, m1_W, m1_b, m2_W, m2_b)

    return pl.pallas_call(
        _node_b_body,
        grid=(n_b,),
        in_specs=[
            pl.BlockSpec((_BN, _H), lambda i: (i, 0)),
            pl.BlockSpec((2, _BN, _HH), lambda i: (0, i, 0)),
            pl.BlockSpec((2, _H), lambda i: (0, 0)),
            pl.BlockSpec((1, _H), lambda i: (0, 0)),
            pl.BlockSpec((1, _H), lambda i: (0, 0)),
            pl.BlockSpec((1, _H), lambda i: (0, 0)),
        ],
        out_specs=pl.BlockSpec((2, _BN, _HH), lambda i: (0, i, 0)),
        out_shape=jax.ShapeDtypeStruct((2, _N, _HH), _F32),
    )(t2, h, sums, gn_w, gn_b, gn_ms)


def _final_body(h_ref, bcol_ref, brow_ref, gx_ref, g1_ref, g1b_ref, g2_ref,
                g2b_ref, gp_ref, gpb_ref, lnw_ref, lnb_ref, h1_ref, h1b_ref,
                h2_ref, h2b_ref, out_ref):
    hf = jnp.concatenate([h_ref[0], h_ref[1]], axis=1)
    bm = bcol_ref[...] == lax.broadcasted_iota(jnp.int32, (_N, _G), 1)
    bmT = brow_ref[...] == lax.broadcasted_iota(jnp.int32, (_G, _N), 0)
    bf = bm.astype(_F32)
    bfT = bmT.astype(_F32)

    cnt = jnp.dot(bfT, jnp.ones((_N, 1), _F32), preferred_element_type=_F32)
    hsum = jnp.dot(bfT, hf, preferred_element_type=_F32)
    h_mean = hsum / jnp.maximum(cnt, 1.0)

    g1 = g1_ref[...]
    gh = jnp.dot(h_ref[0], g1[:_HH], preferred_element_type=_F32)
    gh = gh + jnp.dot(h_ref[1], g1[_HH:], preferred_element_type=_F32) + g1b_ref[...]
    gh = jnp.maximum(gh, 0.0)
    gate = jnp.dot(gh, g2_ref[...], preferred_element_type=_F32) + g2b_ref[...]

    gm = jnp.max(jnp.where(bm, gate, -jnp.inf), axis=0, keepdims=True)
    gm = jnp.where(jnp.isfinite(gm), gm, 0.0)
    gmb = jnp.sum(bf * gm, axis=1, keepdims=True)
    eg = jnp.exp(gate - gmb)
    den = jnp.dot(bfT, eg, preferred_element_type=_F32)
    denb = jnp.dot(bf, den, preferred_element_type=_F32)
    alpha = eg / (denb + 1e-16)
    h_attn = jnp.dot(bfT, alpha * hf, preferred_element_type=_F32)

    g = jnp.dot(gx_ref[...], gp_ref[...], preferred_element_type=_F32) + gpb_ref[...]
    g = jnp.maximum(g, 0.0)

    zc = jnp.concatenate([h_mean, h_attn, g], axis=1)
    mu = jnp.mean(zc, axis=1, keepdims=True)
    var = jnp.mean((zc - mu) ** 2, axis=1, keepdims=True)
    zcn = lnw_ref[...] * (zc - mu) / jnp.sqrt(var + 1e-5) + lnb_ref[...]

    z1 = jnp.dot(zcn, h1_ref[...], preferred_element_type=_F32) + h1b_ref[...]
    z1 = jnp.maximum(z1, 0.0)
    out_ref[...] = jnp.dot(z1, h2_ref[...], preferred_element_type=_F32) + h2b_ref[...]


def _tc_final(h, batch_col, batch_row, global_x, g1_W, g1_b, g2_W, g2_b,
              gp_W, gp_b, ln_w, ln_b, h1_W, h1_b, h2_W, h2_b):
    return pl.pallas_call(
        _final_body,
        out_shape=jax.ShapeDtypeStruct((_G, 64), _F32),
    )(h, batch_col, batch_row, global_x, g1_W, g1_b, g2_W, g2_b,
      gp_W, gp_b, ln_w, ln_b, h1_W, h1_b, h2_W, h2_b)


def kernel(x, edge_index, edge_attr, batch, global_x, node_W, node_b, edge_W,
           edge_b, lin_W, lin_b, m1_W, m1_b, m2_W, m2_b, gn_w, gn_b, gn_ms,
           g1_W, g1_b, g2_W, g2_b, gp_W, gp_b, ln_w, ln_b, h1_W, h1_b, h2_W,
           h2_b):
    src = edge_index[0]
    dst = edge_index[1]
    src2 = jnp.concatenate([src, src + _N])

    h = _tc_h0(x, node_W, jnp.reshape(node_b, (1, _H)))
    lin_W_bf = lin_W.astype(jnp.bfloat16)
    eb_r = jnp.reshape(edge_b, (1, _H))

    for i in range(_L):
        lea = _tc_lea(edge_attr, edge_W, eb_r, lin_W_bf[i],
                      jnp.reshape(lin_b[i], (1, _H)))
        lea2 = jnp.reshape(lea, (2 * _E, _HH))
        h2 = jnp.reshape(h, (2 * _N, _HH))
        agg2 = _sc_layer(h2, lea2, src2, dst)
        agg = jnp.reshape(agg2, (2, _NP, _HH))[:, :_N]
        h = _tc_node(agg, h, m1_W[i], jnp.reshape(m1_b[i], (1, _H)),
                     m2_W[i], jnp.reshape(m2_b[i], (1, _H)),
                     jnp.reshape(gn_w[i], (1, _H)), jnp.reshape(gn_b[i], (1, _H)),
                     jnp.reshape(gn_ms[i], (1, _H)))

    out = _tc_final(h, jnp.reshape(batch, (_N, 1)), jnp.reshape(batch, (1, _N)),
                    global_x, g1_W, jnp.reshape(g1_b, (1, _HH)), g2_W,
                    jnp.reshape(g2_b, (1, 1)), gp_W, jnp.reshape(gp_b, (1, _H)),
                    jnp.reshape(ln_w, (1, 3 * _H)), jnp.reshape(ln_b, (1, 3 * _H)),
                    h1_W, jnp.reshape(h1_b, (1, _H)), h2_W, jnp.reshape(h2_b, (1, 64)))
    return out


# speedup vs baseline: 4.3506x; 1.4381x over previous
import functools

import jax
import jax.numpy as jnp
from jax import lax
from jax.experimental import pallas as pl
from jax.experimental.pallas import tpu as pltpu
from jax.experimental.pallas import tpu_sc as plsc

_N = 10000
_E = 320000
_DF = 128
_DE = 16
_H = 256
_HH = 128
_L = 4
_G = 16

_F32 = jnp.float32


def _h0_body(x_ref, w_ref, b_ref, out_ref):
    h = jnp.dot(x_ref[...], w_ref[...], preferred_element_type=_F32) + b_ref[...]
    h = jnp.maximum(h, 0.0)
    out_ref[0] = h[:, :_HH]
    out_ref[1] = h[:, _HH:]


def _tc_h0(x, node_W, node_b):
    return pl.pallas_call(
        _h0_body,
        out_shape=jax.ShapeDtypeStruct((2, _N, _HH), _F32),
    )(x, node_W, node_b)


_BE = 2560


def _lea_body(eattr_ref, ew_ref, eb_ref, lw_ref, lb_ref, out_ref):
    ea = jnp.dot(eattr_ref[...], ew_ref[...], preferred_element_type=_F32) + eb_ref[...]
    ea = jnp.maximum(ea, 0.0)
    lea = jnp.dot(ea.astype(jnp.bfloat16), lw_ref[...],
                  preferred_element_type=_F32) + lb_ref[...]
    out_ref[0] = lea[:, :_HH]
    out_ref[1] = lea[:, _HH:]


def _tc_lea(edge_attr, edge_W, edge_b, lin_W_bf, lin_b):
    n_e = _E // _BE
    return pl.pallas_call(
        _lea_body,
        grid=(n_e,),
        in_specs=[
            pl.BlockSpec((_BE, _DE), lambda e: (e, 0)),
            pl.BlockSpec((_DE, _H), lambda e: (0, 0)),
            pl.BlockSpec((1, _H), lambda e: (0, 0)),
            pl.BlockSpec((_H, _H), lambda e: (0, 0)),
            pl.BlockSpec((1, _H), lambda e: (0, 0)),
        ],
        out_specs=pl.BlockSpec((2, _BE, _HH), lambda e: (0, e, 0)),
        out_shape=jax.ShapeDtypeStruct((2, _E, _HH), _F32),
    )(edge_attr, edge_W, edge_b, lin_W_bf, lin_b)


_K = 160
_EPT = _E // 16
_NCH = _EPT // _K
_NP = 10240
_NPT = _NP // 16


def _sc_body(h2, lea2, src2, dst, agg2, aggsp,
             msg0, msg1, sidx0, sidx1, didx0, didx1,
             sem_ix0, sem_ix1, sem_lea0, sem_lea1,
             sem_g0, sem_g1, sem_s0, sem_s1):
    c = lax.axis_index("c")
    s = lax.axis_index("s")
    ebase = s * _EPT

    msgs = (msg0, msg1)
    sidxs = (sidx0, sidx1)
    didxs = (didx0, didx1)
    sem_ix = (sem_ix0, sem_ix1)
    sem_lea = (sem_lea0, sem_lea1)
    sem_g = (sem_g0, sem_g1)
    sem_s = (sem_s0, sem_s1)

    def issue_inputs(n, b):
        base = ebase + n * _K
        pltpu.async_copy(src2.at[pl.ds(c * _E + base, _K)], sidxs[b], sem_ix[b])
        pltpu.async_copy(dst.at[pl.ds(base, _K)], didxs[b], sem_ix[b])
        pltpu.async_copy(lea2.at[pl.ds(c * _E + base, _K)], msgs[b], sem_lea[b])

    def wait_inputs(b):
        pltpu.make_async_copy(src2.at[pl.ds(0, _K)], sidxs[b], sem_ix[b]).wait()
        pltpu.make_async_copy(dst.at[pl.ds(0, _K)], didxs[b], sem_ix[b]).wait()
        pltpu.make_async_copy(lea2.at[pl.ds(0, _K)], msgs[b], sem_lea[b]).wait()

    def issue_gather(b):
        pltpu.async_copy(h2.at[sidxs[b]], msgs[b], sem_g[b], add=True)

    def wait_gather(b):
        pltpu.make_async_copy(h2.at[sidxs[b]], msgs[b], sem_g[b]).wait()

    def issue_scatter(b):
        pltpu.async_copy(msgs[b], aggsp.at[didxs[b]], sem_s[b], add=True)

    def wait_scatter(b):
        pltpu.make_async_copy(msgs[b], aggsp.at[didxs[b]], sem_s[b]).wait()

    def relu(b):
        mref = msgs[b]

        @pl.loop(0, _K)
        def _relu_row(e):
            for j in range(_HH // 16):
                v = mref[e, pl.ds(j * 16, 16)]
                mref[e, pl.ds(j * 16, 16)] = jnp.maximum(v, 0.0)

    issue_inputs(0, 0)

    @pl.loop(0, _K)
    def _zero_row(e):
        for j in range(_HH // 16):
            msg1[e, pl.ds(j * 16, 16)] = jnp.zeros((16,), _F32)

    for off in range(0, _NPT - _K + 1, _K):
        pltpu.sync_copy(msg1.at[pl.ds(0, _K)], aggsp.at[pl.ds(s * _NPT + off, _K)])
    plsc.subcore_barrier()

    wait_inputs(0)
    issue_gather(0)

    @pl.loop(0, (_NCH - 1) // 2)
    def _pair(m):
        a = 2 * m + 1

        @pl.when(m > 0)
        def _():
            wait_scatter(1)

        issue_inputs(a, 1)
        wait_gather(0)
        relu(0)
        issue_scatter(0)
        wait_inputs(1)
        issue_gather(1)
        wait_scatter(0)
        issue_inputs(a + 1, 0)
        wait_gather(1)
        relu(1)
        issue_scatter(1)
        wait_inputs(0)
        issue_gather(0)

    wait_gather(0)
    relu(0)
    issue_scatter(0)
    wait_scatter(0)
    wait_scatter(1)
    plsc.subcore_barrier()
    pltpu.sync_copy(aggsp.at[pl.ds(s * _NPT, _NPT)],
                    agg2.at[pl.ds(c * _NP + s * _NPT, _NPT)])


def _sc_layer(h2, lea2, src2, dst):
    mesh = plsc.VectorSubcoreMesh(core_axis_name="c", subcore_axis_name="s",
                                  num_cores=2, num_subcores=16)
    fn = pl.kernel(
        _sc_body,
        out_type=jax.ShapeDtypeStruct((2 * _NP, _HH), _F32),
        mesh=mesh,
        scratch_types=[
            pltpu.VMEM_SHARED((_NP, _HH), _F32),
            pltpu.VMEM((_K, _HH), _F32),
            pltpu.VMEM((_K, _HH), _F32),
            pltpu.VMEM((_K,), jnp.int32),
            pltpu.VMEM((_K,), jnp.int32),
            pltpu.VMEM((_K,), jnp.int32),
            pltpu.VMEM((_K,), jnp.int32),
            pltpu.SemaphoreType.DMA,
            pltpu.SemaphoreType.DMA,
            pltpu.SemaphoreType.DMA,
            pltpu.SemaphoreType.DMA,
            pltpu.SemaphoreType.DMA,
            pltpu.SemaphoreType.DMA,
            pltpu.SemaphoreType.DMA,
            pltpu.SemaphoreType.DMA,
        ],
    )
    return fn(h2, lea2, src2, dst)


_BN = 2000


def _node_a_body(agg_ref, h_ref, m1_ref, b1_ref, m2_ref, b2_ref, t2_ref, sums_ref):
    i = pl.program_id(0)
    m1 = m1_ref[...]
    t_lo = agg_ref[0] + h_ref[0]
    t_hi = agg_ref[1] + h_ref[1]
    r1 = jnp.dot(t_lo, m1[:_HH], preferred_element_type=_F32)
    r1 = r1 + jnp.dot(t_hi, m1[_HH:], preferred_element_type=_F32) + b1_ref[...]
    r1 = jnp.maximum(r1, 0.0)
    t2 = jnp.dot(r1, m2_ref[...], preferred_element_type=_F32) + b2_ref[...]
    t2_ref[...] = t2
    ssum = jnp.sum(t2, axis=0, keepdims=True)
    ssq = jnp.sum(t2 * t2, axis=0, keepdims=True)
    both = jnp.concatenate([ssum, ssq], axis=0)

    @pl.when(i == 0)
    def _():
        sums_ref[...] = both

    @pl.when(i > 0)
    def _():
        sums_ref[...] = sums_ref[...] + both


def _node_b_body(t2_ref, h_ref, sums_ref, gnw_ref, gnb_ref, gnms_ref, out_ref):
    inv_n = 1.0 / _N
    mu = sums_ref[0:1] * inv_n
    m2s = sums_ref[1:2] * inv_n
    ms = gnms_ref[...]
    var = m2s - mu * mu * ms * (2.0 - ms)
    cen = t2_ref[...] - mu * ms
    t3 = gnw_ref[...] * cen * lax.rsqrt(var + 1e-5) + gnb_ref[...]
    t3 = jnp.maximum(t3, 0.0)
    out_ref[0] = h_ref[0] + t3[:, :_HH]
    out_ref[1] = h_ref[1] + t3[:, _HH:]


def _tc_node(agg, h, m1_W, m1_b, m2_W, m2_b, gn_w, gn_b, gn_ms):
    n_b = _N // _BN
    t2, sums = pl.pallas_call(
        _node_a_body,
        grid=(n_b,),
        in_specs=[
            pl.BlockSpec((2, _BN, _HH), lambda i: (0, i, 0)),
            pl.BlockSpec((2, _BN, _HH), lambda i: (0, i, 0)),
            pl.BlockSpec((_H, _H), lambda i: (0, 0)),
            pl.BlockSpec((1, _H), lambda i: (0, 0)),
            pl.BlockSpec((_H, _H), lambda i: (0, 0)),
            pl.BlockSpec((1, _H), lambda i: (0, 0)),
        ],
        out_specs=[
            pl.BlockSpec((_BN, _H), lambda i: (i, 0)),
            pl.BlockSpec((2, _H), lambda i: (0, 0)),
        ],
        out_shape=[
            jax.ShapeDtypeStruct((_N, _H), _F32),
            jax.ShapeDtypeStruct((2, _H), _F32),
        ],
    )(agg, h, m1_W, m1_b, m2_W, m2_b)

    return pl.pallas_call(
        _node_b_body,
        grid=(n_b,),
        in_specs=[
            pl.BlockSpec((_BN, _H), lambda i: (i, 0)),
            pl.BlockSpec((2, _BN, _HH), lambda i: (0, i, 0)),
            pl.BlockSpec((2, _H), lambda i: (0, 0)),
            pl.BlockSpec((1, _H), lambda i: (0, 0)),
            pl.BlockSpec((1, _H), lambda i: (0, 0)),
            pl.BlockSpec((1, _H), lambda i: (0, 0)),
        ],
        out_specs=pl.BlockSpec((2, _BN, _HH), lambda i: (0, i, 0)),
        out_shape=jax.ShapeDtypeStruct((2, _N, _HH), _F32),
    )(t2, h, sums, gn_w, gn_b, gn_ms)


def _final_body(h_ref, bcol_ref, brow_ref, gx_ref, g1_ref, g1b_ref, g2_ref,
                g2b_ref, gp_ref, gpb_ref, lnw_ref, lnb_ref, h1_ref, h1b_ref,
                h2_ref, h2b_ref, out_ref):
    hf = jnp.concatenate([h_ref[0], h_ref[1]], axis=1)
    bm = bcol_ref[...] == lax.broadcasted_iota(jnp.int32, (_N, _G), 1)
    bmT = brow_ref[...] == lax.broadcasted_iota(jnp.int32, (_G, _N), 0)
    bf = bm.astype(_F32)
    bfT = bmT.astype(_F32)

    cnt = jnp.dot(bfT, jnp.ones((_N, 1), _F32), preferred_element_type=_F32)
    hsum = jnp.dot(bfT, hf, preferred_element_type=_F32)
    h_mean = hsum / jnp.maximum(cnt, 1.0)

    g1 = g1_ref[...]
    gh = jnp.dot(h_ref[0], g1[:_HH], preferred_element_type=_F32)
    gh = gh + jnp.dot(h_ref[1], g1[_HH:], preferred_element_type=_F32) + g1b_ref[...]
    gh = jnp.maximum(gh, 0.0)
    gate = jnp.dot(gh, g2_ref[...], preferred_element_type=_F32) + g2b_ref[...]

    gm = jnp.max(jnp.where(bm, gate, -jnp.inf), axis=0, keepdims=True)
    gm = jnp.where(jnp.isfinite(gm), gm, 0.0)
    gmb = jnp.sum(bf * gm, axis=1, keepdims=True)
    eg = jnp.exp(gate - gmb)
    den = jnp.dot(bfT, eg, preferred_element_type=_F32)
    denb = jnp.dot(bf, den, preferred_element_type=_F32)
    alpha = eg / (denb + 1e-16)
    h_attn = jnp.dot(bfT, alpha * hf, preferred_element_type=_F32)

    g = jnp.dot(gx_ref[...], gp_ref[...], preferred_element_type=_F32) + gpb_ref[...]
    g = jnp.maximum(g, 0.0)

    zc = jnp.concatenate([h_mean, h_attn, g], axis=1)
    mu = jnp.mean(zc, axis=1, keepdims=True)
    var = jnp.mean((zc - mu) ** 2, axis=1, keepdims=True)
    zcn = lnw_ref[...] * (zc - mu) / jnp.sqrt(var + 1e-5) + lnb_ref[...]

    z1 = jnp.dot(zcn, h1_ref[...], preferred_element_type=_F32) + h1b_ref[...]
    z1 = jnp.maximum(z1, 0.0)
    out_ref[...] = jnp.dot(z1, h2_ref[...], preferred_element_type=_F32) + h2b_ref[...]


def _tc_final(h, batch_col, batch_row, global_x, g1_W, g1_b, g2_W, g2_b,
              gp_W, gp_b, ln_w, ln_b, h1_W, h1_b, h2_W, h2_b):
    return pl.pallas_call(
        _final_body,
        out_shape=jax.ShapeDtypeStruct((_G, 64), _F32),
    )(h, batch_col, batch_row, global_x, g1_W, g1_b, g2_W, g2_b,
      gp_W, gp_b, ln_w, ln_b, h1_W, h1_b, h2_W, h2_b)


def kernel(x, edge_index, edge_attr, batch, global_x, node_W, node_b, edge_W,
           edge_b, lin_W, lin_b, m1_W, m1_b, m2_W, m2_b, gn_w, gn_b, gn_ms,
           g1_W, g1_b, g2_W, g2_b, gp_W, gp_b, ln_w, ln_b, h1_W, h1_b, h2_W,
           h2_b):
    src = edge_index[0]
    dst = edge_index[1]
    src2 = jnp.concatenate([src, src + _N])

    h = _tc_h0(x, node_W, jnp.reshape(node_b, (1, _H)))
    lin_W_bf = lin_W.astype(jnp.bfloat16)
    eb_r = jnp.reshape(edge_b, (1, _H))

    for i in range(_L):
        lea = _tc_lea(edge_attr, edge_W, eb_r, lin_W_bf[i],
                      jnp.reshape(lin_b[i], (1, _H)))
        lea2 = jnp.reshape(lea, (2 * _E, _HH))
        h2 = jnp.reshape(h, (2 * _N, _HH))
        agg2 = _sc_layer(h2, lea2, src2, dst)
        agg = jnp.reshape(agg2, (2, _NP, _HH))[:, :_N]
        h = _tc_node(agg, h, m1_W[i], jnp.reshape(m1_b[i], (1, _H)),
                     m2_W[i], jnp.reshape(m2_b[i], (1, _H)),
                     jnp.reshape(gn_w[i], (1, _H)), jnp.reshape(gn_b[i], (1, _H)),
                     jnp.reshape(gn_ms[i], (1, _H)))

    out = _tc_final(h, jnp.reshape(batch, (_N, 1)), jnp.reshape(batch, (1, _N)),
                    global_x, g1_W, jnp.reshape(g1_b, (1, _HH)), g2_W,
                    jnp.reshape(g2_b, (1, 1)), gp_W, jnp.reshape(gp_b, (1, _H)),
                    jnp.reshape(ln_w, (1, 3 * _H)), jnp.reshape(ln_b, (1, 3 * _H)),
                    h1_W, jnp.reshape(h1_b, (1, _H)), h2_W, jnp.reshape(h2_b, (1, 64)))
    return out
